# baseline (speedup 1.0000x reference)
import functools

import jax
import jax.numpy as jnp
from jax import lax
from jax.experimental import pallas as pl
from jax.experimental.pallas import tpu as pltpu
from jax.experimental.pallas import tpu_sc as plsc

_LANES = 16
_POS_PER_CHUNK = 16


def kernel(x, pe_table):
    B, S, D = x.shape

    info = plsc.get_sparse_core_info()
    NC, NS = info.num_cores, info.num_subcores
    NW = NC * NS
    pos_per_w = S // NW
    P = _POS_PER_CHUNK
    n_chunks = pos_per_w // P

    @functools.partial(
        pl.kernel,
        mesh=plsc.VectorSubcoreMesh(core_axis_name="c", subcore_axis_name="s"),
        out_type=jax.ShapeDtypeStruct((B, S, D), jnp.float32),
        scratch_types=[
            pltpu.VMEM((B, P, D), jnp.float32),
            pltpu.VMEM((B, P, D), jnp.float32),
            pltpu.VMEM((P, D), jnp.float32),
            pltpu.VMEM((P, D), jnp.float32),
            pltpu.SemaphoreType.DMA,
            pltpu.SemaphoreType.DMA,
            pltpu.SemaphoreType.DMA,
            pltpu.SemaphoreType.DMA,
            pltpu.SemaphoreType.DMA,
            pltpu.SemaphoreType.DMA,
        ],
    )
    def sc_add(x_hbm, pe_hbm, out_hbm, xa, xb, pea, peb,
               sem_xa, sem_xb, sem_pea, sem_peb, sem_oa, sem_ob):
        wid = lax.axis_index("s") * NC + lax.axis_index("c")
        base_pos = wid * pos_per_w

        xbufs = (xa, xb)
        pebufs = (pea, peb)
        xsems = (sem_xa, sem_xb)
        pesems = (sem_pea, sem_peb)
        osems = (sem_oa, sem_ob)

        handles = {}

        def pos0(ci):
            return base_pos + ci * P

        handles[("x", 0)] = pltpu.async_copy(
            x_hbm.at[:, pl.ds(pos0(0), P), :], xbufs[0], xsems[0])
        handles[("pe", 0)] = pltpu.async_copy(
            pe_hbm.at[pl.ds(pos0(0), P), :], pebufs[0], pesems[0])

        for ci in range(n_chunks):
            xi = ci % 2

            if ci + 1 < n_chunks:
                ni = (ci + 1) % 2
                if ("o", ci - 1) in handles:
                    handles[("o", ci - 1)].wait()
                handles[("x", ci + 1)] = pltpu.async_copy(
                    x_hbm.at[:, pl.ds(pos0(ci + 1), P), :],
                    xbufs[ni], xsems[ni])
                handles[("pe", ci + 1)] = pltpu.async_copy(
                    pe_hbm.at[pl.ds(pos0(ci + 1), P), :],
                    pebufs[ni], pesems[ni])

            handles[("x", ci)].wait()
            handles[("pe", ci)].wait()

            xbuf = xbufs[xi]
            pebuf = pebufs[xi]

            for b in range(B):
                def row_body(r, carry):
                    @plsc.parallel_loop(0, D, step=_LANES, unroll=8)
                    def slice_body(c):
                        sl = pl.ds(c, _LANES)
                        xbuf[b, r, sl] = xbuf[b, r, sl] + pebuf[r, sl]

                    return carry

                lax.fori_loop(0, P, row_body, 0)

            handles[("o", ci)] = pltpu.async_copy(
                xbuf, out_hbm.at[:, pl.ds(pos0(ci), P), :], osems[xi])

        handles[("o", n_chunks - 2)].wait()
        handles[("o", n_chunks - 1)].wait()

    out = sc_add(x, pe_table)
    return out


# speedup vs baseline: 1.0333x; 1.0333x over previous
import functools

import jax
import jax.numpy as jnp
from jax import lax
from jax.experimental import pallas as pl
from jax.experimental.pallas import tpu as pltpu
from jax.experimental.pallas import tpu_sc as plsc

_LANES = 16
_POS_PER_CHUNK = 32


def kernel(x, pe_table):
    B, S, D = x.shape
    x2 = x.reshape(B * S, D)

    info = plsc.get_sparse_core_info()
    NC, NS = info.num_cores, info.num_subcores
    NW = NC * NS
    pos_per_w = S // NW
    P = _POS_PER_CHUNK
    n_chunks = pos_per_w // P
    n_steps = n_chunks * B

    @functools.partial(
        pl.kernel,
        mesh=plsc.VectorSubcoreMesh(core_axis_name="c", subcore_axis_name="s"),
        out_type=jax.ShapeDtypeStruct((B * S, D), jnp.float32),
        scratch_types=[
            pltpu.VMEM((P, D), jnp.float32),
            pltpu.VMEM((P, D), jnp.float32),
            pltpu.VMEM((P, D), jnp.float32),
            pltpu.VMEM((P, D), jnp.float32),
            pltpu.VMEM((P, D), jnp.float32),
            pltpu.SemaphoreType.DMA,
            pltpu.SemaphoreType.DMA,
            pltpu.SemaphoreType.DMA,
            pltpu.SemaphoreType.DMA,
            pltpu.SemaphoreType.DMA,
            pltpu.SemaphoreType.DMA,
            pltpu.SemaphoreType.DMA,
            pltpu.SemaphoreType.DMA,
        ],
    )
    def sc_add(x_hbm, pe_hbm, out_hbm, xa, xb, xc, pea, peb,
               sem_xa, sem_xb, sem_xc, sem_pea, sem_peb,
               sem_oa, sem_ob, sem_oc):
        wid = lax.axis_index("s") * NC + lax.axis_index("c")
        base_pos = wid * pos_per_w

        xbufs = (xa, xb, xc)
        pebufs = (pea, peb)
        xsems = (sem_xa, sem_xb, sem_xc)
        pesems = (sem_pea, sem_peb)
        osems = (sem_oa, sem_ob, sem_oc)

        handles = {}

        def pos0(ci):
            return base_pos + ci * P

        def x_row(k):
            ci, b = divmod(k, B)
            return b * S + pos0(ci)

        handles[("x", 0)] = pltpu.async_copy(
            x_hbm.at[pl.ds(x_row(0), P), :], xbufs[0], xsems[0])
        handles[("x", 1)] = pltpu.async_copy(
            x_hbm.at[pl.ds(x_row(1), P), :], xbufs[1], xsems[1])
        handles[("pe", 0)] = pltpu.async_copy(
            pe_hbm.at[pl.ds(pos0(0), P), :], pebufs[0], pesems[0])

        for k in range(n_steps):
            ci, b = divmod(k, B)
            xi = k % 3
            pi = ci % 2

            if k + 2 < n_steps:
                ni = (k + 2) % 3
                if ("o", k - 1) in handles:
                    handles[("o", k - 1)].wait()
                handles[("x", k + 2)] = pltpu.async_copy(
                    x_hbm.at[pl.ds(x_row(k + 2), P), :], xbufs[ni], xsems[ni])

            if b == 0 and ci + 1 < n_chunks:
                npi = (ci + 1) % 2
                handles[("pe", ci + 1)] = pltpu.async_copy(
                    pe_hbm.at[pl.ds(pos0(ci + 1), P), :],
                    pebufs[npi], pesems[npi])

            handles[("x", k)].wait()
            if b == 0:
                handles[("pe", ci)].wait()

            xbuf = xbufs[xi]
            pebuf = pebufs[pi]

            def row_body(r, carry):
                @plsc.parallel_loop(0, D, step=_LANES, unroll=8)
                def slice_body(c):
                    sl = pl.ds(c, _LANES)
                    xbuf[r, sl] = xbuf[r, sl] + pebuf[r, sl]

                return carry

            lax.fori_loop(0, P, row_body, 0)

            handles[("o", k)] = pltpu.async_copy(
                xbuf, out_hbm.at[pl.ds(x_row(k), P), :], osems[xi])

        handles[("o", n_steps - 3)].wait()
        handles[("o", n_steps - 2)].wait()
        handles[("o", n_steps - 1)].wait()

    out = sc_add(x2, pe_table)
    return out.reshape(B, S, D)


# speedup vs baseline: 1.1933x; 1.1548x over previous
import functools

import jax
import jax.numpy as jnp
from jax import lax
from jax.experimental import pallas as pl
from jax.experimental.pallas import tpu as pltpu
from jax.experimental.pallas import tpu_sc as plsc

_LANES = 16
_POS_PER_CHUNK = 16


def kernel(x, pe_table):
    B, S, D = x.shape

    info = plsc.get_sparse_core_info()
    NC, NS = info.num_cores, info.num_subcores
    NW = NC * NS
    pos_per_w = S // NW
    P = _POS_PER_CHUNK
    n_chunks = pos_per_w // P

    @functools.partial(
        pl.kernel,
        mesh=plsc.VectorSubcoreMesh(core_axis_name="c", subcore_axis_name="s"),
        out_type=jax.ShapeDtypeStruct((B, S, D), jnp.float32),
        scratch_types=[
            pltpu.VMEM((B, P, D), jnp.float32),
            pltpu.VMEM((B, P, D), jnp.float32),
            pltpu.VMEM((P, D), jnp.float32),
            pltpu.VMEM((P, D), jnp.float32),
            pltpu.SemaphoreType.DMA,
            pltpu.SemaphoreType.DMA,
            pltpu.SemaphoreType.DMA,
            pltpu.SemaphoreType.DMA,
            pltpu.SemaphoreType.DMA,
            pltpu.SemaphoreType.DMA,
        ],
    )
    def sc_add(x_hbm, pe_hbm, out_hbm, xa, xb, pea, peb,
               sem_xa, sem_xb, sem_pea, sem_peb, sem_oa, sem_ob):
        wid = lax.axis_index("s") * NC + lax.axis_index("c")
        base_pos = wid * pos_per_w

        xbufs = (xa, xb)
        pebufs = (pea, peb)
        xsems = (sem_xa, sem_xb)
        pesems = (sem_pea, sem_peb)
        osems = (sem_oa, sem_ob)

        handles = {}

        def pos0(ci):
            return base_pos + ci * P

        handles[("x", 0)] = pltpu.async_copy(
            x_hbm.at[:, pl.ds(pos0(0), P), :], xbufs[0], xsems[0])
        handles[("pe", 0)] = pltpu.async_copy(
            pe_hbm.at[pl.ds(pos0(0), P), :], pebufs[0], pesems[0])

        for ci in range(n_chunks):
            xi = ci % 2

            if ci + 1 < n_chunks:
                ni = (ci + 1) % 2
                if ("o", ci - 1) in handles:
                    handles[("o", ci - 1)].wait()
                handles[("x", ci + 1)] = pltpu.async_copy(
                    x_hbm.at[:, pl.ds(pos0(ci + 1), P), :],
                    xbufs[ni], xsems[ni])
                handles[("pe", ci + 1)] = pltpu.async_copy(
                    pe_hbm.at[pl.ds(pos0(ci + 1), P), :],
                    pebufs[ni], pesems[ni])

            handles[("x", ci)].wait()
            handles[("pe", ci)].wait()

            xbuf = xbufs[xi]
            pebuf = pebufs[xi]

            def row_body(r, carry):
                @plsc.parallel_loop(0, D, step=_LANES, unroll=4)
                def slice_body(c):
                    sl = pl.ds(c, _LANES)
                    pv = pebuf[r, sl]
                    for b in range(B):
                        xbuf[b, r, sl] = xbuf[b, r, sl] + pv

                return carry

            lax.fori_loop(0, P, row_body, 0)

            handles[("o", ci)] = pltpu.async_copy(
                xbuf, out_hbm.at[:, pl.ds(pos0(ci), P), :], osems[xi])

        handles[("o", n_chunks - 2)].wait()
        handles[("o", n_chunks - 1)].wait()

    out = sc_add(x, pe_table)
    return out


# speedup vs baseline: 1.1985x; 1.0043x over previous
import functools

import jax
import jax.numpy as jnp
from jax import lax
from jax.experimental import pallas as pl
from jax.experimental.pallas import tpu as pltpu
from jax.experimental.pallas import tpu_sc as plsc

_LANES = 16
_POS_PER_CHUNK = 16


def kernel(x, pe_table):
    B, S, D = x.shape

    info = plsc.get_sparse_core_info()
    NC, NS = info.num_cores, info.num_subcores
    NW = NC * NS
    pos_per_w = S // NW
    P = _POS_PER_CHUNK
    n_chunks = pos_per_w // P

    @functools.partial(
        pl.kernel,
        mesh=plsc.VectorSubcoreMesh(core_axis_name="c", subcore_axis_name="s"),
        out_type=jax.ShapeDtypeStruct((B, S, D), jnp.float32),
        scratch_types=[
            pltpu.VMEM((B, P, D), jnp.float32),
            pltpu.VMEM((B, P, D), jnp.float32),
            pltpu.VMEM((P, D), jnp.float32),
            pltpu.VMEM((P, D), jnp.float32),
            pltpu.SemaphoreType.DMA,
            pltpu.SemaphoreType.DMA,
            pltpu.SemaphoreType.DMA,
            pltpu.SemaphoreType.DMA,
            pltpu.SemaphoreType.DMA,
            pltpu.SemaphoreType.DMA,
        ],
    )
    def sc_add(x_hbm, pe_hbm, out_hbm, xa, xb, pea, peb,
               sem_xa, sem_xb, sem_pea, sem_peb, sem_oa, sem_ob):
        wid = lax.axis_index("s") * NC + lax.axis_index("c")
        base_pos = wid * pos_per_w

        xbufs = (xa, xb)
        pebufs = (pea, peb)
        xsems = (sem_xa, sem_xb)
        pesems = (sem_pea, sem_peb)
        osems = (sem_oa, sem_ob)

        handles = {}

        def pos0(ci):
            return base_pos + ci * P

        handles[("x", 0)] = pltpu.async_copy(
            x_hbm.at[:, pl.ds(pos0(0), P), :], xbufs[0], xsems[0])
        handles[("pe", 0)] = pltpu.async_copy(
            pe_hbm.at[pl.ds(pos0(0), P), :], pebufs[0], pesems[0])

        for ci in range(n_chunks):
            xi = ci % 2

            if ci + 1 < n_chunks:
                ni = (ci + 1) % 2
                if ("o", ci - 1) in handles:
                    handles[("o", ci - 1)].wait()
                handles[("x", ci + 1)] = pltpu.async_copy(
                    x_hbm.at[:, pl.ds(pos0(ci + 1), P), :],
                    xbufs[ni], xsems[ni])
                handles[("pe", ci + 1)] = pltpu.async_copy(
                    pe_hbm.at[pl.ds(pos0(ci + 1), P), :],
                    pebufs[ni], pesems[ni])

            handles[("x", ci)].wait()
            handles[("pe", ci)].wait()

            xbuf = xbufs[xi]
            pebuf = pebufs[xi]

            def row_body(r, carry):
                @plsc.parallel_loop(0, D, step=_LANES, unroll=8)
                def slice_body(c):
                    sl = pl.ds(c, _LANES)
                    pv = pebuf[r, sl]
                    for b in range(B):
                        xbuf[b, r, sl] = xbuf[b, r, sl] + pv

                return carry

            lax.fori_loop(0, P, row_body, 0)

            handles[("o", ci)] = pltpu.async_copy(
                xbuf, out_hbm.at[:, pl.ds(pos0(ci), P), :], osems[xi])

        handles[("o", n_chunks - 2)].wait()
        handles[("o", n_chunks - 1)].wait()

    out = sc_add(x, pe_table)
    return out
